# baseline (speedup 1.0000x reference)
import functools

import jax
import jax.numpy as jnp
from jax.experimental import pallas as pl

_LANES = 128
_ROW_BLOCK = 4096


def _threefry_bits(m):
    k1 = jnp.uint32(0)
    k2 = jnp.uint32(42)
    ks2 = k1 ^ k2 ^ jnp.uint32(0x1BD11BDA)

    x0 = jnp.full_like(m, k1)
    x1 = m + k2

    def rounds(x0, x1, rots, a0, a1, c):
        for r in rots:
            x0 = x0 + x1
            x1 = x0 ^ ((x1 << jnp.uint32(r)) | (x1 >> jnp.uint32(32 - r)))
        return x0 + a0, x1 + a1 + jnp.uint32(c)

    rot_a = (13, 15, 26, 6)
    rot_b = (17, 29, 16, 24)
    x0, x1 = rounds(x0, x1, rot_a, k2, ks2, 1)
    x0, x1 = rounds(x0, x1, rot_b, ks2, k1, 2)
    x0, x1 = rounds(x0, x1, rot_a, k1, k2, 3)
    x0, x1 = rounds(x0, x1, rot_b, k2, ks2, 4)
    x0, x1 = rounds(x0, x1, rot_a, ks2, k1, 5)
    return x0 ^ x1


def _uniform_from_bits(bits):
    tiny = jnp.float32(1.1754944e-38)
    fb = (bits >> jnp.uint32(9)) | jnp.uint32(0x3F800000)
    f = jax.lax.bitcast_convert_type(fb, jnp.float32) - jnp.float32(1.0)
    return jnp.maximum(tiny, f * (jnp.float32(1.0) - tiny) + tiny)


def _fused_body(x_ref, w_ref, b_ref, dec_ref, pp_ref):
    rows, _ = x_ref.shape
    s = rows // _LANES

    o = jax.lax.dot_general(
        x_ref[...], w_ref[...],
        dimension_numbers=(((1,), (0,)), ((), ())),
        preferred_element_type=jnp.float32,
    )
    o = o + b_ref[0, 0]
    o = o.reshape(s, _LANES)

    p = jax.nn.sigmoid(o)
    p0 = jnp.float32(1.0) - p
    lo = jnp.float32(1e-20)
    hi = jnp.float32(1.0)
    logit0 = jnp.log(jnp.clip(p0, lo, hi))
    logit1 = jnp.log(jnp.clip(p, lo, hi))

    base = pl.program_id(0).astype(jnp.uint32) * jnp.uint32(rows)
    row = (base
           + jax.lax.broadcasted_iota(jnp.uint32, (s, _LANES), 0) * jnp.uint32(_LANES)
           + jax.lax.broadcasted_iota(jnp.uint32, (s, _LANES), 1))
    m0 = row * jnp.uint32(2)
    m1 = m0 + jnp.uint32(1)

    g0 = -jnp.log(-jnp.log(_uniform_from_bits(_threefry_bits(m0))))
    g1 = -jnp.log(-jnp.log(_uniform_from_bits(_threefry_bits(m1))))

    dec_ref[...] = (logit1 + g1 > logit0 + g0).astype(jnp.int32)
    pp_ref[...] = jnp.stack([p0, p], axis=-1).reshape(s, 2 * _LANES)


@functools.partial(jax.jit, static_argnums=())
def kernel(x, W, b, num_steps):
    n, d = x.shape
    r = _ROW_BLOCK if n % _ROW_BLOCK == 0 else n
    sb = r // _LANES
    s = n // _LANES
    b2 = b.reshape(1, 1)
    wt = W.reshape(d, 1)

    dec, pp = pl.pallas_call(
        _fused_body,
        grid=(n // r,),
        in_specs=[
            pl.BlockSpec((r, d), lambda i: (i, 0)),
            pl.BlockSpec((d, 1), lambda i: (0, 0)),
            pl.BlockSpec((1, 1), lambda i: (0, 0)),
        ],
        out_specs=[
            pl.BlockSpec((sb, _LANES), lambda i: (i, 0)),
            pl.BlockSpec((sb, 2 * _LANES), lambda i: (i, 0)),
        ],
        out_shape=[
            jax.ShapeDtypeStruct((s, _LANES), jnp.int32),
            jax.ShapeDtypeStruct((s, 2 * _LANES), jnp.float32),
        ],
    )(x, wt, b2)

    steps = n // 128
    decision = dec.reshape(-1, steps)
    probs = pp.reshape(-1, steps, 2)
    return (decision, probs)


# speedup vs baseline: 1.5763x; 1.5763x over previous
import functools

import jax
import jax.numpy as jnp
from jax.experimental import pallas as pl

_LANES = 128
_ROW_BLOCK = 4096


def _threefry_bits(m):
    k1 = jnp.uint32(0)
    k2 = jnp.uint32(42)
    ks2 = k1 ^ k2 ^ jnp.uint32(0x1BD11BDA)

    x0 = jnp.full_like(m, k1)
    x1 = m + k2

    def rounds(x0, x1, rots, a0, a1, c):
        for r in rots:
            x0 = x0 + x1
            x1 = x0 ^ ((x1 << jnp.uint32(r)) | (x1 >> jnp.uint32(32 - r)))
        return x0 + a0, x1 + a1 + jnp.uint32(c)

    rot_a = (13, 15, 26, 6)
    rot_b = (17, 29, 16, 24)
    x0, x1 = rounds(x0, x1, rot_a, k2, ks2, 1)
    x0, x1 = rounds(x0, x1, rot_b, ks2, k1, 2)
    x0, x1 = rounds(x0, x1, rot_a, k1, k2, 3)
    x0, x1 = rounds(x0, x1, rot_b, k2, ks2, 4)
    x0, x1 = rounds(x0, x1, rot_a, ks2, k1, 5)
    return x0 ^ x1


def _uniform_from_bits(bits):
    tiny = jnp.float32(1.1754944e-38)
    fb = (bits >> jnp.uint32(9)) | jnp.uint32(0x3F800000)
    f = jax.lax.bitcast_convert_type(fb, jnp.float32) - jnp.float32(1.0)
    return jnp.maximum(tiny, f * (jnp.float32(1.0) - tiny) + tiny)


def _fused_body(x_ref, w_ref, b_ref, dec_ref, p0_ref, p1_ref):
    rows, _ = x_ref.shape
    s = rows // _LANES

    o = jax.lax.dot_general(
        x_ref[...], w_ref[...],
        dimension_numbers=(((1,), (0,)), ((), ())),
        preferred_element_type=jnp.float32,
    )
    o = o + b_ref[0, 0]
    o = o.reshape(s, _LANES)

    p = jax.nn.sigmoid(o)
    p0 = jnp.float32(1.0) - p
    lo = jnp.float32(1e-20)
    hi = jnp.float32(1.0)
    logit0 = jnp.log(jnp.clip(p0, lo, hi))
    logit1 = jnp.log(jnp.clip(p, lo, hi))

    base = pl.program_id(0).astype(jnp.uint32) * jnp.uint32(rows)
    row = (base
           + jax.lax.broadcasted_iota(jnp.uint32, (s, _LANES), 0) * jnp.uint32(_LANES)
           + jax.lax.broadcasted_iota(jnp.uint32, (s, _LANES), 1))
    m0 = row * jnp.uint32(2)
    m1 = m0 + jnp.uint32(1)

    g0 = -jnp.log(-jnp.log(_uniform_from_bits(_threefry_bits(m0))))
    g1 = -jnp.log(-jnp.log(_uniform_from_bits(_threefry_bits(m1))))

    dec_ref[...] = (logit1 + g1 > logit0 + g0).astype(jnp.int32)
    p0_ref[...] = p0
    p1_ref[...] = p


@functools.partial(jax.jit, static_argnums=())
def kernel(x, W, b, num_steps):
    n, d = x.shape
    r = _ROW_BLOCK if n % _ROW_BLOCK == 0 else n
    sb = r // _LANES
    s = n // _LANES
    b2 = b.reshape(1, 1)
    wt = W.reshape(d, 1)

    dec, p0, p1 = pl.pallas_call(
        _fused_body,
        grid=(n // r,),
        in_specs=[
            pl.BlockSpec((r, d), lambda i: (i, 0)),
            pl.BlockSpec((d, 1), lambda i: (0, 0)),
            pl.BlockSpec((1, 1), lambda i: (0, 0)),
        ],
        out_specs=[
            pl.BlockSpec((sb, _LANES), lambda i: (i, 0)),
            pl.BlockSpec((sb, _LANES), lambda i: (i, 0)),
            pl.BlockSpec((sb, _LANES), lambda i: (i, 0)),
        ],
        out_shape=[
            jax.ShapeDtypeStruct((s, _LANES), jnp.int32),
            jax.ShapeDtypeStruct((s, _LANES), jnp.float32),
            jax.ShapeDtypeStruct((s, _LANES), jnp.float32),
        ],
    )(x, wt, b2)

    steps = n // 128
    decision = dec.reshape(-1, steps)
    probs = jnp.stack([p0.reshape(-1), p1.reshape(-1)], axis=-1)
    probs = probs.reshape(-1, steps, 2)
    return (decision, probs)


# speedup vs baseline: 1.5792x; 1.0018x over previous
import functools

import jax
import jax.numpy as jnp
from jax.experimental import pallas as pl

_LANES = 128
_ROW_BLOCK = 4096


def _threefry_bits(m):
    k1 = jnp.uint32(0)
    k2 = jnp.uint32(42)
    ks2 = k1 ^ k2 ^ jnp.uint32(0x1BD11BDA)

    x0 = jnp.full_like(m, k1)
    x1 = m + k2

    def rounds(x0, x1, rots, a0, a1, c):
        for r in rots:
            x0 = x0 + x1
            x1 = x0 ^ ((x1 << jnp.uint32(r)) | (x1 >> jnp.uint32(32 - r)))
        return x0 + a0, x1 + a1 + jnp.uint32(c)

    rot_a = (13, 15, 26, 6)
    rot_b = (17, 29, 16, 24)
    x0, x1 = rounds(x0, x1, rot_a, k2, ks2, 1)
    x0, x1 = rounds(x0, x1, rot_b, ks2, k1, 2)
    x0, x1 = rounds(x0, x1, rot_a, k1, k2, 3)
    x0, x1 = rounds(x0, x1, rot_b, k2, ks2, 4)
    x0, x1 = rounds(x0, x1, rot_a, ks2, k1, 5)
    return x0 ^ x1


def _uniform_from_bits(bits):
    tiny = jnp.float32(1.1754944e-38)
    fb = (bits >> jnp.uint32(9)) | jnp.uint32(0x3F800000)
    f = jax.lax.bitcast_convert_type(fb, jnp.float32) - jnp.float32(1.0)
    return jnp.maximum(tiny, f * (jnp.float32(1.0) - tiny) + tiny)


def _fused_body(xa_ref, xb_ref, w_ref, b_ref, dec_ref, p0_ref, p1_ref):
    rows, half = xa_ref.shape
    s = rows // _LANES

    def part(x_ref, j):
        return jax.lax.dot_general(
            x_ref[...], w_ref[j * half:(j + 1) * half, :],
            dimension_numbers=(((1,), (0,)), ((), ())),
            preferred_element_type=jnp.float32,
        )

    o = part(xa_ref, 0) + part(xb_ref, 1)
    o = o + b_ref[0, 0]
    o = o.reshape(s, _LANES)

    p = jax.nn.sigmoid(o)
    p0 = jnp.float32(1.0) - p
    lo = jnp.float32(1e-20)
    hi = jnp.float32(1.0)
    logit0 = jnp.log(jnp.clip(p0, lo, hi))
    logit1 = jnp.log(jnp.clip(p, lo, hi))

    base = pl.program_id(0).astype(jnp.uint32) * jnp.uint32(rows)
    row = (base
           + jax.lax.broadcasted_iota(jnp.uint32, (s, _LANES), 0) * jnp.uint32(_LANES)
           + jax.lax.broadcasted_iota(jnp.uint32, (s, _LANES), 1))
    m0 = row * jnp.uint32(2)
    m1 = m0 + jnp.uint32(1)

    g0 = -jnp.log(-jnp.log(_uniform_from_bits(_threefry_bits(m0))))
    g1 = -jnp.log(-jnp.log(_uniform_from_bits(_threefry_bits(m1))))

    dec_ref[...] = (logit1 + g1 > logit0 + g0).astype(jnp.int32)
    p0_ref[...] = p0
    p1_ref[...] = p


@functools.partial(jax.jit, static_argnums=())
def kernel(x, W, b, num_steps):
    n, d = x.shape
    r = _ROW_BLOCK if n % _ROW_BLOCK == 0 else n
    sb = r // _LANES
    s = n // _LANES
    b2 = b.reshape(1, 1)
    wt = W.reshape(d, 1)

    dec, p0, p1 = pl.pallas_call(
        _fused_body,
        grid=(n // r,),
        in_specs=[
            pl.BlockSpec((r, d // 2), lambda i: (i, 0)),
            pl.BlockSpec((r, d // 2), lambda i: (i, 1)),
            pl.BlockSpec((d, 1), lambda i: (0, 0)),
            pl.BlockSpec((1, 1), lambda i: (0, 0)),
        ],
        out_specs=[
            pl.BlockSpec((sb, _LANES), lambda i: (i, 0)),
            pl.BlockSpec((sb, _LANES), lambda i: (i, 0)),
            pl.BlockSpec((sb, _LANES), lambda i: (i, 0)),
        ],
        out_shape=[
            jax.ShapeDtypeStruct((s, _LANES), jnp.int32),
            jax.ShapeDtypeStruct((s, _LANES), jnp.float32),
            jax.ShapeDtypeStruct((s, _LANES), jnp.float32),
        ],
    )(x, x, wt, b2)

    steps = n // 128
    decision = dec.reshape(-1, steps)
    probs = jnp.stack([p0.reshape(-1), p1.reshape(-1)], axis=-1)
    probs = probs.reshape(-1, steps, 2)
    return (decision, probs)
